# baseline (speedup 1.0000x reference)
import jax
import jax.numpy as jnp
import numpy as np
from jax import lax
from jax.experimental import pallas as pl
from jax.experimental.pallas import tpu as pltpu
from jax.experimental.pallas import tpu_sc as plsc

B, W, H, C, M = 4, 64, 64, 20, 100
WH = W * H
MT = 8
BIG_I = 1 << 30
NPAIR = B * M
PPT = 8
NPH = 200
LANES = 16
NCHUNK = WH // LANES
UNROLL = 8


def _cost_body(pred_ref, cls_true_ref, loc_true_ref, total_ref, clsid_ref):
    pt = pred_ref[0, :C]
    ct = jnp.equal(cls_true_ref[0], 1).astype(jnp.float32)

    neg = 0.75 * (pt * pt) * (-jnp.log(1.0 - pt))
    pos = 0.25 * ((1.0 - pt) * (1.0 - pt)) * (-jnp.log(pt + 1e-08))
    d = pos - neg

    parts = []
    for mt in range(0, M, MT):
        ct_t = ct[mt:mt + MT]
        acc = ct_t[:, 0][:, None] * d[0, :][None, :]
        for c in range(1, C):
            acc = acc + ct_t[:, c][:, None] * d[c, :][None, :]
        parts.append(acc)
    cls_loss = jnp.concatenate(parts, axis=0)

    lp = pred_ref[0, C:] / jnp.float32(W)
    lt = loc_true_ref[0]

    b1_ymin = lp[0, :][None, :]
    b1_xmin = lp[1, :][None, :]
    b1_ymax = lp[2, :][None, :]
    b1_xmax = lp[3, :][None, :]
    b2_ymin = lt[:, 0][:, None]
    b2_xmin = lt[:, 1][:, None]
    b2_ymax = lt[:, 2][:, None]
    b2_xmax = lt[:, 3][:, None]

    zero = jnp.float32(0.0)
    b1_area = (jnp.maximum(zero, b1_xmax - b1_xmin)
               * jnp.maximum(zero, b1_ymax - b1_ymin))
    b2_area = (jnp.maximum(zero, b2_xmax - b2_xmin)
               * jnp.maximum(zero, b2_ymax - b2_ymin))

    i_w = jnp.maximum(zero, jnp.minimum(b1_xmax, b2_xmax)
                      - jnp.maximum(b1_xmin, b2_xmin))
    i_h = jnp.maximum(zero, jnp.minimum(b1_ymax, b2_ymax)
                      - jnp.maximum(b1_ymin, b2_ymin))
    i_area = i_w * i_h
    union = b1_area + b2_area - i_area
    iou = jnp.where(union > 0, i_area / jnp.where(union > 0, union, 1.0), 0.0)

    e_w = jnp.maximum(zero, jnp.maximum(b1_xmax, b2_xmax)
                      - jnp.minimum(b1_xmin, b2_xmin))
    e_h = jnp.maximum(zero, jnp.maximum(b1_ymax, b2_ymax)
                      - jnp.minimum(b1_ymin, b2_ymin))
    e_area = e_w * e_h
    giou = iou - jnp.where(e_area > 0,
                           (e_area - union) / jnp.where(e_area > 0, e_area, 1.0),
                           0.0)
    giou_l = 1.0 - giou

    reg = (jnp.abs(b2_ymin - b1_ymin) + jnp.abs(b2_xmin - b1_xmin)
           + jnp.abs(b2_ymax - b1_ymax) + jnp.abs(b2_xmax - b1_xmax))

    total_ref[0] = 2.0 * cls_loss + 5.0 * reg + 2.0 * giou_l

    ct_max = jnp.max(ct, axis=1, keepdims=True)
    iota_c = lax.broadcasted_iota(jnp.int32, (M, C), 1)
    clsid_ref[0, 0, :] = jnp.min(jnp.where(ct == ct_max, iota_c, C), axis=1)


HB = 2


def _tc_cost_half(pred_t, cls_true, loc_true, off):
    return pl.pallas_call(
        _cost_body,
        grid=(HB,),
        in_specs=[
            pl.BlockSpec((1, C + 4, WH), lambda b: (b + off, 0, 0)),
            pl.BlockSpec((1, M, C), lambda b: (b + off, 0, 0)),
            pl.BlockSpec((1, M, 4), lambda b: (b + off, 0, 0)),
        ],
        out_specs=[
            pl.BlockSpec((1, M, WH), lambda b: (b, 0, 0)),
            pl.BlockSpec((1, 1, M), lambda b: (b, 0, 0)),
        ],
        out_shape=[
            jax.ShapeDtypeStruct((HB, M, WH), jnp.float32),
            jax.ShapeDtypeStruct((HB, 1, M), jnp.int32),
        ],
    )(pred_t, cls_true, loc_true)


def _sc_argmin_kernel(total_hbm, out_hbm, rows_v, res_v, sem0):
    info = plsc.get_sparse_core_info()
    wid = lax.axis_index("s") * info.num_cores + lax.axis_index("c")
    p0 = jnp.minimum(wid * PPT, NPH - PPT)
    pltpu.async_copy(total_hbm.at[pl.ds(p0, PPT)], rows_v, sem0).wait()

    iota16 = lax.iota(jnp.int32, LANES)
    res = jnp.zeros((LANES,), jnp.int32)
    for j in range(PPT):

        def step(i, carry):
            minv, mini, idxv = carry
            for u in range(UNROLL):
                off = (i * UNROLL + u) * LANES
                v = rows_v[j, pl.ds(off, LANES)]
                upd = v < minv
                minv = jnp.where(upd, v, minv)
                mini = jnp.where(upd, idxv, mini)
                idxv = idxv + jnp.full((LANES,), LANES, jnp.int32)
            return (minv, mini, idxv)

        init = (jnp.full((LANES,), jnp.inf, jnp.float32),
                jnp.zeros((LANES,), jnp.int32), iota16)
        minv, mini, _ = lax.fori_loop(0, NCHUNK // UNROLL, step, init)

        gmin = minv
        for k in (1, 2, 4, 8):
            gmin = jnp.minimum(gmin, jnp.take(gmin, iota16 ^ k))
        cand = jnp.where(minv == gmin, mini, jnp.full((LANES,), BIG_I, jnp.int32))
        for k in (1, 2, 4, 8):
            cand = jnp.minimum(cand, jnp.take(cand, iota16 ^ k))
        res = jnp.where(iota16 == jnp.full((LANES,), j, jnp.int32), cand, res)

    res_v[...] = res
    pltpu.sync_copy(res_v, out_hbm.at[wid])


def _sc_argmin(total):
    mesh = plsc.VectorSubcoreMesh(core_axis_name="c", subcore_axis_name="s")
    run = pl.kernel(
        _sc_argmin_kernel,
        mesh=mesh,
        out_type=jax.ShapeDtypeStruct((32, LANES), jnp.int32),
        scratch_types=[
            pltpu.VMEM((PPT, WH), jnp.float32),
            pltpu.VMEM((LANES,), jnp.int32),
            pltpu.SemaphoreType.DMA,
        ],
    )
    return run(total.reshape(NPH, WH))


def kernel(cls_pred, loc_pred, cls_true, loc_true, reg_mask):
    del reg_mask
    pred = jnp.concatenate((cls_pred.reshape(B, WH, C),
                            loc_pred.reshape(B, WH, 4)), axis=-1)
    pred_t = pred.transpose(0, 2, 1)

    total0, cid0 = _tc_cost_half(pred_t, cls_true, loc_true, 0)
    total1, cid1 = _tc_cost_half(pred_t, cls_true, loc_true, HB)
    am0 = _sc_argmin(total0)
    am1 = _sc_argmin(total1)

    p = np.arange(NPH)
    flat = jnp.asarray((p // PPT) * LANES + p % PPT, dtype=jnp.int32)
    cid = jnp.concatenate((cid0, cid1), axis=0)
    am = jnp.concatenate((am0.reshape(32 * LANES)[flat],
                          am1.reshape(32 * LANES)[flat])).reshape(B, M)[..., None]
    cid = cid.reshape(B, M)[..., None]
    batch = jnp.tile(jnp.arange(B, dtype=jnp.int32)[:, None], (1, M))[..., None]
    return jnp.concatenate((batch, am, cid), axis=-1)


# speedup vs baseline: 2.0783x; 2.0783x over previous
import jax
import jax.numpy as jnp
from jax import lax
from jax.experimental import pallas as pl

B, W, H, C, M = 4, 64, 64, 20, 100
WH = W * H
MT = 8
BIG_I = 1 << 30


def _matcher_body(pred_ref, cls_true_ref, loc_true_ref, out_ref):
    pt = pred_ref[0, :C]
    ct = jnp.equal(cls_true_ref[0], 1).astype(jnp.float32)

    neg = 0.75 * (pt * pt) * (-jnp.log(1.0 - pt))
    pos = 0.25 * ((1.0 - pt) * (1.0 - pt)) * (-jnp.log(pt + 1e-08))
    d = pos - neg

    parts = []
    for mt in range(0, M, MT):
        ct_t = ct[mt:mt + MT]
        acc = ct_t[:, 0][:, None] * d[0, :][None, :]
        for c in range(1, C):
            acc = acc + ct_t[:, c][:, None] * d[c, :][None, :]
        parts.append(acc)
    cls_loss = jnp.concatenate(parts, axis=0)

    lp = pred_ref[0, C:] / jnp.float32(W)
    lt = loc_true_ref[0]

    b1_ymin = lp[0, :][None, :]
    b1_xmin = lp[1, :][None, :]
    b1_ymax = lp[2, :][None, :]
    b1_xmax = lp[3, :][None, :]
    b2_ymin = lt[:, 0][:, None]
    b2_xmin = lt[:, 1][:, None]
    b2_ymax = lt[:, 2][:, None]
    b2_xmax = lt[:, 3][:, None]

    zero = jnp.float32(0.0)
    b1_area = (jnp.maximum(zero, b1_xmax - b1_xmin)
               * jnp.maximum(zero, b1_ymax - b1_ymin))
    b2_area = (jnp.maximum(zero, b2_xmax - b2_xmin)
               * jnp.maximum(zero, b2_ymax - b2_ymin))

    i_w = jnp.maximum(zero, jnp.minimum(b1_xmax, b2_xmax)
                      - jnp.maximum(b1_xmin, b2_xmin))
    i_h = jnp.maximum(zero, jnp.minimum(b1_ymax, b2_ymax)
                      - jnp.maximum(b1_ymin, b2_ymin))
    i_area = i_w * i_h
    union = b1_area + b2_area - i_area
    iou = jnp.where(union > 0, i_area / jnp.where(union > 0, union, 1.0), 0.0)

    e_w = jnp.maximum(zero, jnp.maximum(b1_xmax, b2_xmax)
                      - jnp.minimum(b1_xmin, b2_xmin))
    e_h = jnp.maximum(zero, jnp.maximum(b1_ymax, b2_ymax)
                      - jnp.minimum(b1_ymin, b2_ymin))
    e_area = e_w * e_h
    giou = iou - jnp.where(e_area > 0,
                           (e_area - union) / jnp.where(e_area > 0, e_area, 1.0),
                           0.0)
    giou_l = 1.0 - giou

    reg = (jnp.abs(b2_ymin - b1_ymin) + jnp.abs(b2_xmin - b1_xmin)
           + jnp.abs(b2_ymax - b1_ymax) + jnp.abs(b2_xmax - b1_xmax))

    total = 2.0 * cls_loss + 5.0 * reg + 2.0 * giou_l

    iota = lax.broadcasted_iota(jnp.int32, (M, WH), 1)
    cmin = jnp.min(total, axis=1)
    am = jnp.min(jnp.where(total == cmin[:, None], iota, BIG_I), axis=1)

    maxv = jnp.max(ct, axis=1, keepdims=True)
    iota_c = lax.broadcasted_iota(jnp.int32, (M, C), 1)
    cid = jnp.min(jnp.where(ct == maxv, iota_c, C), axis=1)

    bcol = jnp.full((M,), pl.program_id(0), jnp.int32)
    out_ref[0] = jnp.concatenate(
        (bcol[:, None], am[:, None], cid[:, None]), axis=1)


def kernel(cls_pred, loc_pred, cls_true, loc_true, reg_mask):
    del reg_mask
    pred_t = jnp.concatenate((cls_pred.reshape(B, WH, C),
                              loc_pred.reshape(B, WH, 4)),
                             axis=-1).transpose(0, 2, 1)
    return pl.pallas_call(
        _matcher_body,
        grid=(B,),
        in_specs=[
            pl.BlockSpec((1, C + 4, WH), lambda b: (b, 0, 0)),
            pl.BlockSpec((1, M, C), lambda b: (b, 0, 0)),
            pl.BlockSpec((1, M, 4), lambda b: (b, 0, 0)),
        ],
        out_specs=pl.BlockSpec((1, M, 3), lambda b: (b, 0, 0)),
        out_shape=jax.ShapeDtypeStruct((B, M, 3), jnp.int32),
    )(pred_t, cls_true, loc_true)
